# baseline (device time: 20501 ns/iter reference)
import jax
import jax.numpy as jnp
from jax import lax
from jax.experimental import pallas as pl
from jax.experimental.pallas import tpu as pltpu

N_CHUNKS = 8


def kernel(A, B):
    m, k = A.shape
    _, n = B.shape
    chunk_m = m // N_CHUNKS

    def body(a_ref, b_ref, out_ref, send_buf, recv_buf, send_sems, recv_sems):
        my_x = lax.axis_index("x")
        my_y = lax.axis_index("y")
        peer = (1 - my_x, my_y)

        barrier_sem = pltpu.get_barrier_semaphore()
        pl.semaphore_signal(
            barrier_sem, inc=1, device_id=peer,
            device_id_type=pl.DeviceIdType.MESH,
        )

        send_buf[...] = jnp.zeros((m, n), jnp.bfloat16)
        pl.semaphore_wait(barrier_sem, 1)

        rdmas = []
        for i in range(N_CHUNKS):
            sl = pl.ds(i * chunk_m, chunk_m)
            rdma = pltpu.make_async_remote_copy(
                src_ref=send_buf.at[sl, :],
                dst_ref=recv_buf.at[sl, :],
                send_sem=send_sems.at[i],
                recv_sem=recv_sems.at[i],
                device_id=peer,
                device_id_type=pl.DeviceIdType.MESH,
            )
            rdma.start()
            rdmas.append(rdma)

        for i in range(N_CHUNKS):
            sl = pl.ds(i * chunk_m, chunk_m)
            rdmas[i].wait_recv()
            out_ref[sl, :] = (
                send_buf[sl, :].astype(jnp.float32)
                + recv_buf[sl, :].astype(jnp.float32)
            )

        for i in range(N_CHUNKS):
            rdmas[i].wait_send()

    return pl.pallas_call(
        body,
        out_shape=jax.ShapeDtypeStruct((m, n), jnp.float32),
        in_specs=[
            pl.BlockSpec(memory_space=pltpu.VMEM),
            pl.BlockSpec(memory_space=pltpu.VMEM),
        ],
        out_specs=pl.BlockSpec(memory_space=pltpu.VMEM),
        scratch_shapes=[
            pltpu.VMEM((m, n), jnp.bfloat16),
            pltpu.VMEM((m, n), jnp.bfloat16),
            pltpu.SemaphoreType.DMA((N_CHUNKS,)),
            pltpu.SemaphoreType.DMA((N_CHUNKS,)),
        ],
        compiler_params=pltpu.CompilerParams(collective_id=0),
    )(A, B)


# device time: 18799 ns/iter; 1.0905x vs baseline; 1.0905x over previous
import jax
import jax.numpy as jnp
from jax import lax
from jax.experimental import pallas as pl
from jax.experimental.pallas import tpu as pltpu

N_B_CHUNKS = 3


def kernel(A, B):
    m, k = A.shape
    _, n = B.shape
    half_m = m // 2
    chunk_n = n // N_B_CHUNKS

    def body(a_ref, b_ref, out_ref,
             a_stage, b_stage, a_recv, b_recv,
             ax_send_sem, ax_recv_sem, ay_send_sem, ay_recv_sem,
             b_send_sems, b_recv_sems):
        my_x = lax.axis_index("x")
        my_y = lax.axis_index("y")
        x_peer = (1 - my_x, my_y)
        y_peer = (my_x, 1 - my_y)

        barrier_sem = pltpu.get_barrier_semaphore()
        for nbr in (x_peer, y_peer):
            pl.semaphore_signal(
                barrier_sem, inc=1, device_id=nbr,
                device_id_type=pl.DeviceIdType.MESH,
            )

        my_half = pl.ds(my_y * half_m, half_m)
        a_stage[...] = a_ref[my_half, :].astype(jnp.bfloat16)

        pl.semaphore_wait(barrier_sem, 2)

        ax_rdma = pltpu.make_async_remote_copy(
            src_ref=a_stage,
            dst_ref=a_recv.at[my_half, :],
            send_sem=ax_send_sem,
            recv_sem=ax_recv_sem,
            device_id=x_peer,
            device_id_type=pl.DeviceIdType.MESH,
        )
        ax_rdma.start()

        b_stage[...] = b_ref[...].astype(jnp.bfloat16)
        b_rdmas = []
        for j in range(N_B_CHUNKS):
            nc = pl.ds(j * chunk_n, chunk_n)
            rdma = pltpu.make_async_remote_copy(
                src_ref=b_stage.at[:, nc],
                dst_ref=b_recv.at[:, nc],
                send_sem=b_send_sems.at[j],
                recv_sem=b_recv_sems.at[j],
                device_id=x_peer,
                device_id_type=pl.DeviceIdType.MESH,
            )
            rdma.start()
            b_rdmas.append(rdma)

        a_loc = a_ref[...].astype(jnp.bfloat16)

        def local_chunk(j):
            nc = pl.ds(j * chunk_n, chunk_n)
            out_ref[:, nc] = jax.lax.dot_general(
                a_loc, b_stage[:, nc],
                dimension_numbers=(((1,), (0,)), ((), ())),
                preferred_element_type=jnp.float32,
            )

        local_chunk(0)

        ax_rdma.wait_recv()
        ay_rdma = pltpu.make_async_remote_copy(
            src_ref=a_recv.at[my_half, :],
            dst_ref=a_recv.at[my_half, :],
            send_sem=ay_send_sem,
            recv_sem=ay_recv_sem,
            device_id=y_peer,
            device_id_type=pl.DeviceIdType.MESH,
        )
        ay_rdma.start()

        local_chunk(1)
        local_chunk(2)

        ay_rdma.wait_recv()
        a_peer = a_recv[...]

        for j in range(N_B_CHUNKS):
            nc = pl.ds(j * chunk_n, chunk_n)
            b_rdmas[j].wait_recv()
            out_ref[:, nc] = out_ref[:, nc] + jax.lax.dot_general(
                a_peer, b_recv[:, nc],
                dimension_numbers=(((1,), (0,)), ((), ())),
                preferred_element_type=jnp.float32,
            )

        ax_rdma.wait_send()
        ay_rdma.wait_send()
        for j in range(N_B_CHUNKS):
            b_rdmas[j].wait_send()

    return pl.pallas_call(
        body,
        out_shape=jax.ShapeDtypeStruct((m, n), jnp.float32),
        in_specs=[
            pl.BlockSpec(memory_space=pltpu.VMEM),
            pl.BlockSpec(memory_space=pltpu.VMEM),
        ],
        out_specs=pl.BlockSpec(memory_space=pltpu.VMEM),
        scratch_shapes=[
            pltpu.VMEM((half_m, k), jnp.bfloat16),
            pltpu.VMEM((k, n), jnp.bfloat16),
            pltpu.VMEM((m, k), jnp.bfloat16),
            pltpu.VMEM((k, n), jnp.bfloat16),
            pltpu.SemaphoreType.DMA,
            pltpu.SemaphoreType.DMA,
            pltpu.SemaphoreType.DMA,
            pltpu.SemaphoreType.DMA,
            pltpu.SemaphoreType.DMA((N_B_CHUNKS,)),
            pltpu.SemaphoreType.DMA((N_B_CHUNKS,)),
        ],
        compiler_params=pltpu.CompilerParams(collective_id=0),
    )(A, B)
